# baseline (speedup 1.0000x reference)
import math

import jax
import jax.numpy as jnp
import numpy as np
from jax import lax
from jax.experimental import pallas as pl
from jax.experimental.pallas import tpu as pltpu
from jax.experimental.pallas import tpu_sc as plsc

MIN_D = 0.01
MAX_D = 1000.0
N_POS = 64
N_HEADS = 16
SEQ = 2048
TOTAL = SEQ * SEQ

NC, NS, L = 2, 16, 16
NW = NC * NS
PER_W = TOTAL // NW
CHUNK = 2048
N_CHUNKS = PER_W // CHUNK
GROUPS = CHUNK // L
DROWS = CHUNK // 128
OROWS = CHUNK * N_HEADS // 128

_LO = np.float32(math.log(float(np.float32(MIN_D))))
_HI = np.float32(math.log(float(np.float32(MAX_D))))
_HL = np.float32(_HI - _LO)
_HALF = np.float32(N_POS / 2.0)
_SCALE = np.float32(float(_HALF) / float(_HL))
_LN2 = np.float32(math.log(2.0))

_LN_COEF = [np.float32(c) for c in (
    0.0036622420884668827, -0.0061479173600673676, 0.00818367674946785,
    -0.014340112917125225, 0.026353642344474792, -0.04940934106707573,
    0.09876491874456406, -0.22222137451171875, 0.6666666865348816,
    0.40546509623527527)]


def _bucket_ids(x):
    neg = x < 0.0
    bits = lax.bitcast_convert_type(jnp.abs(x), jnp.int32)
    e = (bits >> 23) - 127
    m = lax.bitcast_convert_type((bits & 0x007FFFFF) | 0x3F800000, jnp.float32)
    t = m - np.float32(1.5)
    ln_m = _LN_COEF[0]
    for coef in _LN_COEF[1:]:
        ln_m = ln_m * t + coef
    ln_a = e.astype(jnp.float32) * _LN2 + ln_m
    u = jnp.minimum(jnp.maximum(ln_a, _LO), _HI)
    u = (u - _LO) * _SCALE
    val = jnp.where(neg, u + (_HALF - np.float32(1.0)),
                    (_HALF - np.float32(1.0)) - u)
    i = val.astype(jnp.int32)
    return jnp.where(i < 0, i + (N_POS + 1), i)


def _body(d_hbm, table_hbm, out_hbm, d_v, rows_v, table_v,
          sem_i0, sem_i1, sem_o0, sem_o1):
    wid = lax.axis_index("s") * NC + lax.axis_index("c")
    sems_i = (sem_i0, sem_i1)
    sems_o = (sem_o0, sem_o1)

    pltpu.sync_copy(table_hbm, table_v)
    iota = lax.iota(jnp.int32, L)

    d_row0 = wid * (PER_W // 128)
    o_row0 = wid * (PER_W * N_HEADS // 128)

    for b in range(2):
        pltpu.async_copy(d_hbm.at[pl.ds(d_row0 + b * DROWS, DROWS)],
                         d_v.at[b], sems_i[b])

    @pl.loop(0, N_CHUNKS, step=2)
    def _chunk(g0):
        for b in range(2):
            c = g0 + b

            pltpu.make_async_copy(
                d_hbm.at[pl.ds(0, DROWS)], d_v.at[b], sems_i[b]).wait()

            @pl.when(c >= 2)
            def _():
                pltpu.make_async_copy(
                    rows_v.at[b], out_hbm.at[pl.ds(0, OROWS)],
                    sems_o[b]).wait()

            @pl.loop(0, GROUPS, unroll=2)
            def _group(v):
                x = d_v[b, v >> 3, pl.ds((v & 7) * L, L)]
                gidx = _bucket_ids(x) * N_HEADS
                for u in range(L):
                    row = gidx[u] + iota
                    val = plsc.load_gather(table_v, [row])
                    rows_v[b, v * 2 + (u >> 3), pl.ds((u & 7) * L, L)] = val

            @pl.when(c + 2 < N_CHUNKS)
            def _():
                pltpu.async_copy(
                    d_hbm.at[pl.ds(d_row0 + (c + 2) * DROWS, DROWS)],
                    d_v.at[b], sems_i[b])

            pltpu.async_copy(
                rows_v.at[b],
                out_hbm.at[pl.ds(o_row0 + c * OROWS, OROWS)],
                sems_o[b])

    for b in range(2):
        pltpu.make_async_copy(
            rows_v.at[b], out_hbm.at[pl.ds(0, OROWS)], sems_o[b]).wait()


@jax.jit
def _run(d2, table_flat):
    mesh = plsc.VectorSubcoreMesh(core_axis_name="c", subcore_axis_name="s")
    return pl.kernel(
        _body,
        out_type=jax.ShapeDtypeStruct((TOTAL * N_HEADS // 128, 128),
                                      jnp.float32),
        mesh=mesh,
        scratch_types=[
            pltpu.VMEM((2, DROWS, 128), jnp.float32),
            pltpu.VMEM((2, OROWS, 128), jnp.float32),
            pltpu.VMEM(((N_POS + 1) * N_HEADS,), jnp.float32),
            pltpu.SemaphoreType.DMA,
            pltpu.SemaphoreType.DMA,
            pltpu.SemaphoreType.DMA,
            pltpu.SemaphoreType.DMA,
        ],
        compiler_params=pltpu.CompilerParams(
            use_tc_tiling_on_sc=True, needs_layout_passes=False
        ),
    )(d2, table_flat)


def kernel(d_mat, embeddings_table):
    out = _run(d_mat.reshape(TOTAL // 128, 128), embeddings_table.reshape(-1))
    return out.reshape(SEQ, SEQ, N_HEADS)


# speedup vs baseline: 1.3975x; 1.3975x over previous
import math

import jax
import jax.numpy as jnp
import numpy as np
from jax import lax
from jax.experimental import pallas as pl
from jax.experimental.pallas import tpu as pltpu
from jax.experimental.pallas import tpu_sc as plsc

MIN_D = 0.01
MAX_D = 1000.0
N_POS = 64
N_HEADS = 16
SEQ = 2048
TOTAL = SEQ * SEQ

NC, NS, L = 2, 16, 16
NW = NC * NS
PER_W = TOTAL // NW
CHUNK = 2048
N_CHUNKS = PER_W // CHUNK
GROUPS = CHUNK // L
DROWS = CHUNK // 128
OROWS = CHUNK * N_HEADS // 128

_LO = np.float32(math.log(float(np.float32(MIN_D))))
_HI = np.float32(math.log(float(np.float32(MAX_D))))
_HL = np.float32(_HI - _LO)
_HALF = np.float32(N_POS / 2.0)
_SCALE = np.float32(float(_HALF) / float(_HL))
_LN2 = np.float32(math.log(2.0))

_LN_COEF = [np.float32(c) for c in (
    0.0036622420884668827, -0.0061479173600673676, 0.00818367674946785,
    -0.014340112917125225, 0.026353642344474792, -0.04940934106707573,
    0.09876491874456406, -0.22222137451171875, 0.6666666865348816,
    0.40546509623527527)]


def _bucket_ids(x):
    neg = x < 0.0
    bits = lax.bitcast_convert_type(jnp.abs(x), jnp.int32)
    e = (bits >> 23) - 127
    m = lax.bitcast_convert_type((bits & 0x007FFFFF) | 0x3F800000, jnp.float32)
    t = m - np.float32(1.5)
    ln_m = _LN_COEF[0]
    for coef in _LN_COEF[1:]:
        ln_m = ln_m * t + coef
    ln_a = e.astype(jnp.float32) * _LN2 + ln_m
    u = jnp.minimum(jnp.maximum(ln_a, _LO), _HI)
    u = (u - _LO) * _SCALE
    val = jnp.where(neg, u + (_HALF - np.float32(1.0)),
                    (_HALF - np.float32(1.0)) - u)
    i = val.astype(jnp.int32)
    return jnp.where(i < 0, i + (N_POS + 1), i)


def _body(d_hbm, table_hbm, out_hbm, d_v, rows_v, table_v,
          sem_i0, sem_i1, sem_o0, sem_o1):
    wid = lax.axis_index("s") * NC + lax.axis_index("c")
    sems_i = (sem_i0, sem_i1)
    sems_o = (sem_o0, sem_o1)

    pltpu.sync_copy(table_hbm, table_v)
    iota = lax.iota(jnp.int32, L)

    d_row0 = wid * (PER_W // 128)
    o_row0 = wid * (PER_W * N_HEADS // 128)

    for b in range(2):
        pltpu.async_copy(d_hbm.at[pl.ds(d_row0 + b * DROWS, DROWS)],
                         d_v.at[b], sems_i[b])

    @pl.loop(0, N_CHUNKS, step=2)
    def _chunk(g0):
        for b in range(2):
            c = g0 + b

            pltpu.make_async_copy(
                d_hbm.at[pl.ds(0, DROWS)], d_v.at[b], sems_i[b]).wait()

            @pl.when(c >= 2)
            def _():
                pltpu.make_async_copy(
                    rows_v.at[b], out_hbm.at[pl.ds(0, OROWS)],
                    sems_o[b]).wait()

            @plsc.parallel_loop(0, GROUPS, unroll=2)
            def _group(v):
                x = d_v[b, v >> 3, pl.ds((v & 7) * L, L)]
                gidx = _bucket_ids(x) * N_HEADS
                for u in range(L):
                    row = gidx[u] + iota
                    val = plsc.load_gather(table_v, [row])
                    rows_v[b, v * 2 + (u >> 3), pl.ds((u & 7) * L, L)] = val

            @pl.when(c + 2 < N_CHUNKS)
            def _():
                pltpu.async_copy(
                    d_hbm.at[pl.ds(d_row0 + (c + 2) * DROWS, DROWS)],
                    d_v.at[b], sems_i[b])

            pltpu.async_copy(
                rows_v.at[b],
                out_hbm.at[pl.ds(o_row0 + c * OROWS, OROWS)],
                sems_o[b])

    for b in range(2):
        pltpu.make_async_copy(
            rows_v.at[b], out_hbm.at[pl.ds(0, OROWS)], sems_o[b]).wait()


@jax.jit
def _run(d2, table_flat):
    mesh = plsc.VectorSubcoreMesh(core_axis_name="c", subcore_axis_name="s")
    return pl.kernel(
        _body,
        out_type=jax.ShapeDtypeStruct((TOTAL * N_HEADS // 128, 128),
                                      jnp.float32),
        mesh=mesh,
        scratch_types=[
            pltpu.VMEM((2, DROWS, 128), jnp.float32),
            pltpu.VMEM((2, OROWS, 128), jnp.float32),
            pltpu.VMEM(((N_POS + 1) * N_HEADS,), jnp.float32),
            pltpu.SemaphoreType.DMA,
            pltpu.SemaphoreType.DMA,
            pltpu.SemaphoreType.DMA,
            pltpu.SemaphoreType.DMA,
        ],
        compiler_params=pltpu.CompilerParams(
            use_tc_tiling_on_sc=True, needs_layout_passes=False
        ),
    )(d2, table_flat)


def kernel(d_mat, embeddings_table):
    out = _run(d_mat.reshape(TOTAL // 128, 128), embeddings_table.reshape(-1))
    return out.reshape(SEQ, SEQ, N_HEADS)
